# baseline (speedup 1.0000x reference)
import jax
import jax.numpy as jnp
from jax import lax
from jax.experimental import pallas as pl
from jax.experimental.pallas import tpu as pltpu
from jax.experimental.pallas import tpu_sc as plsc

RES_LIST = (64, 128, 256, 512)
NLVL = 4
NPLANE = 3
DF = 8
NOUT = NLVL * NPLANE * DF
NPTS = 1048576
NCORES = 2
NSUB = 16
LANES = 16
NWORK = NCORES * NSUB
PPW = NPTS // NWORK
CH = 128
NCHUNK = PPW // CH
NGRP = CH // LANES
NLP = NLVL * NPLANE
PLANE_DXY = ((0, 2), (1, 0), (2, 1))

YB_LIST = tuple(2048 // r for r in RES_LIST)
INB_MAX = max(8 * (yb + 1) * r for yb, r in zip(YB_LIST, RES_LIST))
PKB_MAX = max((yb + 1) * 4 * r for yb, r in zip(YB_LIST, RES_LIST)) + 16
OB_WORDS = 2048 * 16


def _builder_body(pl0, pl1, pl2, pl3, ot0, ot1, ot2, ot3,
                  inb, pkb, ob, sem):
    planes = (pl0, pl1, pl2, pl3)
    outs = (ot0, ot1, ot2, ot3)
    wid = lax.axis_index("s") * NCORES + lax.axis_index("c")
    iota = lax.iota(jnp.int32, LANES)

    for li in range(NLVL):
        r = RES_LIST[li]
        yb = YB_LIST[li]
        ncy = r // yb
        ntask = NPLANE * ncy
        cr = (yb + 1) * r
        xg_n = r // LANES
        src = planes[li]
        dst = outs[li]

        def task_body(q):
            s = q // ncy
            y0 = (q % ncy) * yb
            for c in range(DF):
                off = ((s * DF + c) * r + y0) * r
                pltpu.async_copy(src.at[pl.ds(off, yb * r)],
                                 inb.at[pl.ds(c * cr, yb * r)], sem)

            @pl.when(y0 + yb < r)
            def _():
                for c in range(DF):
                    off = ((s * DF + c) * r + y0 + yb) * r
                    pltpu.async_copy(src.at[pl.ds(off, r)],
                                     inb.at[pl.ds(c * cr + yb * r, r)], sem)

            for c in range(DF):
                pltpu.make_async_copy(src.at[pl.ds(0, yb * r)],
                                      inb.at[pl.ds(c * cr, yb * r)], sem).wait()

            @pl.when(y0 + yb < r)
            def _():
                for c in range(DF):
                    pltpu.make_async_copy(src.at[pl.ds(0, r)],
                                          inb.at[pl.ds(c * cr + yb * r, r)],
                                          sem).wait()

            def pack_body(g, cy):
                xg = g & (xg_n - 1)
                k = (g >> (xg_n.bit_length() - 1)) & 3
                yy = g >> (xg_n.bit_length() - 1 + 2)
                xo = yy * r + xg * LANES
                a = inb[pl.ds((2 * k) * cr + xo, LANES)]
                b = inb[pl.ds((2 * k + 1) * cr + xo, LANES)]
                w = plsc.bitcast(
                    plsc.pack(a, b, format=plsc.PackFormat.INTERLEAVED),
                    jnp.int32)
                pkb[pl.ds(yy * 4 * r + k * r + xg * LANES, LANES)] = w
                return cy
            lax.fori_loop(0, (yb + 1) * 4 * xg_n, pack_body, 0)

            def il_body(g, cy):
                xg = g & (xg_n - 1)
                k = (g >> (xg_n.bit_length() - 1)) & 3
                yy = g >> (xg_n.bit_length() - 1 + 2)
                po = yy * 4 * r + k * r + xg * LANES
                va = pkb[pl.ds(po, LANES)]
                vb = pkb[pl.ds(po + 1, LANES)]
                vc = pkb[pl.ds(po + 4 * r, LANES)]
                vd = pkb[pl.ds(po + 4 * r + 1, LANES)]
                bi = (yy * r + xg * LANES + iota) * 16
                plsc.store_scatter(ob, [bi + k], va)
                plsc.store_scatter(ob, [bi + (4 + k)], vb)
                plsc.store_scatter(ob, [bi + (8 + k)], vc)
                plsc.store_scatter(ob, [bi + (12 + k)], vd)
                return cy
            lax.fori_loop(0, yb * 4 * xg_n, il_body, 0)

            pltpu.sync_copy(ob.at[pl.ds(0, yb * r * 16)],
                            dst.at[pl.ds((s * r + y0) * r * 16, yb * r * 16)])

        nt = (ntask + NWORK - 1) // NWORK
        for t in range(nt):
            q = wid + t * NWORK
            if (t + 1) * NWORK <= ntask:
                task_body(q)
            else:
                @pl.when(q < ntask)
                def _():
                    task_body(q)


def _sc_build(p0, p1, p2, p3):
    mesh = plsc.VectorSubcoreMesh(core_axis_name="c", subcore_axis_name="s")
    return pl.kernel(
        _builder_body,
        out_type=tuple(
            jax.ShapeDtypeStruct((NPLANE * r * r * 16,), jnp.int32)
            for r in RES_LIST),
        mesh=mesh,
        compiler_params=pltpu.CompilerParams(
            use_tc_tiling_on_sc=False, needs_layout_passes=False),
        scratch_types=[
            pltpu.VMEM((INB_MAX,), jnp.float32),
            pltpu.VMEM((PKB_MAX,), jnp.int32),
            pltpu.VMEM((OB_WORDS,), jnp.int32),
            pltpu.SemaphoreType.DMA,
        ],
    )(p0, p1, p2, p3)


def _tec_body(xf, tbl0, tbl1, tbl2, tbl3, out,
              xs_v, idx_v, w_v, tap_v, out_v,
              xsem0, xsem1, gsem0, gsem1, osem0, osem1):
    tbls = (tbl0, tbl1, tbl2, tbl3)
    xsems = (xsem0, xsem1)
    gsems = (gsem0, gsem1)
    osems = (osem0, osem1)
    wid = lax.axis_index("s") * NCORES + lax.axis_index("c")
    base0 = wid * PPW
    iota = lax.iota(jnp.int32, LANES)
    iota3 = iota * 3
    himask = jnp.full((LANES,), -65536, jnp.int32)

    def fire_xs(ci, b):
        base = base0 + ci * CH
        pltpu.async_copy(xf.at[pl.ds(base * 3, CH * 3)],
                         xs_v.at[b], xsems[b])

    def wait_xs(b):
        pltpu.make_async_copy(xf.at[pl.ds(0, CH * 3)],
                              xs_v.at[b], xsems[b]).wait()

    def compute_idx(b):
        def grp(g, c2):
            g0 = g * LANES
            xd3 = [plsc.load_gather(xs_v.at[b], [iota3 + (g0 * 3 + d)])
                   for d in range(3)]
            for l in range(NLVL):
                r = RES_LIST[l]
                st = []
                for d in range(3):
                    t = xd3[d] * float(r) - 0.5
                    i = (t + 1.0).astype(jnp.int32) - 1
                    w1 = t - i.astype(jnp.float32)
                    w0 = 1.0 - w1
                    lo = i < 0
                    hi = i > r - 2
                    ib = jnp.minimum(jnp.maximum(i, 0), r - 2)
                    wA = jnp.where(lo, w1, jnp.where(hi, 0.0, w0))
                    wB = jnp.where(hi, w0, jnp.where(lo, 0.0, w1))
                    st.append((ib, wA, wB))
                for s in range(NPLANE):
                    dx, dy = PLANE_DXY[s]
                    ibx, wxA, wxB = st[dx]
                    iby, wyA, wyB = st[dy]
                    lp = l * NPLANE + s
                    idx_v[b, lp, pl.ds(g0, LANES)] = iby * r + (ibx + s * r * r)
                    w_v[b, lp, 0, pl.ds(g0, LANES)] = wyA * wxA
                    w_v[b, lp, 1, pl.ds(g0, LANES)] = wyA * wxB
                    w_v[b, lp, 2, pl.ds(g0, LANES)] = wyB * wxA
                    w_v[b, lp, 3, pl.ds(g0, LANES)] = wyB * wxB
            return c2
        lax.fori_loop(0, NGRP, grp, 0)

    def fire_gathers(b):
        for l in range(NLVL):
            for s in range(NPLANE):
                lp = l * NPLANE + s
                pltpu.async_copy(tbls[l].at[idx_v.at[b, lp]],
                                 tap_v.at[b, pl.ds(lp * CH, CH), :],
                                 gsems[b])

    def wait_gathers(b):
        for l in range(NLVL):
            for s in range(NPLANE):
                lp = l * NPLANE + s
                pltpu.make_async_copy(tbls[l].at[idx_v.at[b, lp]],
                                      tap_v.at[b, pl.ds(lp * CH, CH), :],
                                      gsems[b]).wait()

    def combine(b):
        tap = tap_v.at[b]
        outb = out_v.at[b]

        def grp(g, c2):
            g0 = g * LANES
            pt = g0 + iota
            pt96 = pt * NOUT
            for lp in range(NLP):
                w00 = w_v[b, lp, 0, pl.ds(g0, LANES)]
                w01 = w_v[b, lp, 1, pl.ds(g0, LANES)]
                w10 = w_v[b, lp, 2, pl.ds(g0, LANES)]
                w11 = w_v[b, lp, 3, pl.ds(g0, LANES)]
                row = pt + lp * CH
                col0 = (lp // NPLANE) * 24 + (lp % NPLANE) * DF
                for k in range(4):
                    wa = plsc.load_gather(tap, [row, jnp.full((LANES,), k, jnp.int32)])
                    wb = plsc.load_gather(tap, [row, jnp.full((LANES,), k + 4, jnp.int32)])
                    wc = plsc.load_gather(tap, [row, jnp.full((LANES,), k + 8, jnp.int32)])
                    wd = plsc.load_gather(tap, [row, jnp.full((LANES,), k + 12, jnp.int32)])
                    alo = plsc.bitcast(wa << 16, jnp.float32)
                    blo = plsc.bitcast(wb << 16, jnp.float32)
                    clo = plsc.bitcast(wc << 16, jnp.float32)
                    dlo = plsc.bitcast(wd << 16, jnp.float32)
                    ahi = plsc.bitcast(wa & himask, jnp.float32)
                    bhi = plsc.bitcast(wb & himask, jnp.float32)
                    chi = plsc.bitcast(wc & himask, jnp.float32)
                    dhi = plsc.bitcast(wd & himask, jnp.float32)
                    even = alo * w00 + blo * w01 + clo * w10 + dlo * w11
                    odd = ahi * w00 + bhi * w01 + chi * w10 + dhi * w11
                    plsc.store_scatter(outb, [pt96 + (col0 + 2 * k)], even)
                    plsc.store_scatter(outb, [pt96 + (col0 + 2 * k + 1)], odd)
            return c2
        lax.fori_loop(0, NGRP, grp, 0)

    def fire_out(ci, b):
        base = base0 + ci * CH
        pltpu.async_copy(out_v.at[b], out.at[pl.ds(base * NOUT, CH * NOUT)],
                         osems[b])

    def wait_out(b):
        pltpu.make_async_copy(out_v.at[b], out.at[pl.ds(0, CH * NOUT)],
                              osems[b]).wait()

    fire_xs(0, 0)
    fire_xs(1, 1)
    wait_xs(0)
    compute_idx(0)
    fire_gathers(0)

    def pair_body(it, carry):
        for b in range(2):
            ci = it * 2 + b
            b1 = 1 - b
            @pl.when(ci + 1 < NCHUNK)
            def _():
                wait_xs(b1)
                compute_idx(b1)
                fire_gathers(b1)

            @pl.when(ci + 2 < NCHUNK)
            def _():
                fire_xs(ci + 2, b)

            @pl.when(ci >= 2)
            def _():
                wait_out(b)

            wait_gathers(b)
            combine(b)
            fire_out(ci, b)
        return carry

    lax.fori_loop(0, NCHUNK // 2, pair_body, 0)
    wait_out(0)
    wait_out(1)


def _sc_lookup(xf, t0, t1, t2, t3):
    mesh = plsc.VectorSubcoreMesh(core_axis_name="c", subcore_axis_name="s")
    return pl.kernel(
        _tec_body,
        out_type=jax.ShapeDtypeStruct((NPTS * NOUT,), jnp.float32),
        mesh=mesh,
        compiler_params=pltpu.CompilerParams(
            use_tc_tiling_on_sc=False, needs_layout_passes=False),
        scratch_types=[
            pltpu.VMEM((2, 3 * CH), jnp.float32),
            pltpu.VMEM((2, NLP, CH), jnp.int32),
            pltpu.VMEM((2, NLP, 4, CH), jnp.float32),
            pltpu.VMEM((2, NLP * CH, 16), jnp.int32),
            pltpu.VMEM((2, CH * NOUT), jnp.float32),
            pltpu.SemaphoreType.DMA,
            pltpu.SemaphoreType.DMA,
            pltpu.SemaphoreType.DMA,
            pltpu.SemaphoreType.DMA,
            pltpu.SemaphoreType.DMA,
            pltpu.SemaphoreType.DMA,
        ],
    )(xf, t0, t1, t2, t3)


def kernel(x, plane0, plane1, plane2, plane3):
    flat_planes = [p.reshape(-1) for p in (plane0, plane1, plane2, plane3)]
    tbl_flat = _sc_build(*flat_planes)
    tbls = [t.reshape(NPLANE * r * r, 16) for t, r in zip(tbl_flat, RES_LIST)]
    flat = _sc_lookup(x.reshape(-1), *tbls)
    return flat.reshape(NPTS, NOUT)


# speedup vs baseline: 1.2897x; 1.2897x over previous
import jax
import jax.numpy as jnp
from jax import lax
from jax.experimental import pallas as pl
from jax.experimental.pallas import tpu as pltpu
from jax.experimental.pallas import tpu_sc as plsc

RES_LIST = (64, 128, 256, 512)
NLVL = 4
NPLANE = 3
DF = 8
NOUT = NLVL * NPLANE * DF
NPTS = 1048576
NCORES = 2
NSUB = 16
LANES = 16
NWORK = NCORES * NSUB
PPW = NPTS // NWORK
CH = 128
NCHUNK = PPW // CH
NGRP = CH // LANES
NLP = NLVL * NPLANE
PLANE_DXY = ((0, 2), (1, 0), (2, 1))

YB_LIST = tuple(2048 // r for r in RES_LIST)
INB_MAX = max(8 * (yb + 1) * r for yb, r in zip(YB_LIST, RES_LIST))
PKB_MAX = max((yb + 1) * 4 * r for yb, r in zip(YB_LIST, RES_LIST)) + 16
OB_WORDS = 2048 * 16


def _builder_body(pl0, pl1, pl2, pl3, ot0, ot1, ot2, ot3,
                  inb, pkb, ob, sem):
    planes = (pl0, pl1, pl2, pl3)
    outs = (ot0, ot1, ot2, ot3)
    wid = lax.axis_index("s") * NCORES + lax.axis_index("c")
    iota = lax.iota(jnp.int32, LANES)

    for li in range(NLVL):
        r = RES_LIST[li]
        yb = YB_LIST[li]
        ncy = r // yb
        ntask = NPLANE * ncy
        cr = (yb + 1) * r
        xg_n = r // LANES
        src = planes[li]
        dst = outs[li]

        def task_body(q):
            s = q // ncy
            y0 = (q % ncy) * yb
            for c in range(DF):
                off = ((s * DF + c) * r + y0) * r
                pltpu.async_copy(src.at[pl.ds(off, yb * r)],
                                 inb.at[pl.ds(c * cr, yb * r)], sem)

            @pl.when(y0 + yb < r)
            def _():
                for c in range(DF):
                    off = ((s * DF + c) * r + y0 + yb) * r
                    pltpu.async_copy(src.at[pl.ds(off, r)],
                                     inb.at[pl.ds(c * cr + yb * r, r)], sem)

            for c in range(DF):
                pltpu.make_async_copy(src.at[pl.ds(0, yb * r)],
                                      inb.at[pl.ds(c * cr, yb * r)], sem).wait()

            @pl.when(y0 + yb < r)
            def _():
                for c in range(DF):
                    pltpu.make_async_copy(src.at[pl.ds(0, r)],
                                          inb.at[pl.ds(c * cr + yb * r, r)],
                                          sem).wait()

            def pack_body(g, cy):
                xg = g & (xg_n - 1)
                k = (g >> (xg_n.bit_length() - 1)) & 3
                yy = g >> (xg_n.bit_length() - 1 + 2)
                xo = yy * r + xg * LANES
                a = inb[pl.ds((2 * k) * cr + xo, LANES)]
                b = inb[pl.ds((2 * k + 1) * cr + xo, LANES)]
                w = plsc.bitcast(
                    plsc.pack(a, b, format=plsc.PackFormat.INTERLEAVED),
                    jnp.int32)
                pkb[pl.ds(yy * 4 * r + k * r + xg * LANES, LANES)] = w
                return cy
            lax.fori_loop(0, (yb + 1) * 4 * xg_n, pack_body, 0)

            def il_body(g, cy):
                xg = g & (xg_n - 1)
                k = (g >> (xg_n.bit_length() - 1)) & 3
                yy = g >> (xg_n.bit_length() - 1 + 2)
                po = yy * 4 * r + k * r + xg * LANES
                va = pkb[pl.ds(po, LANES)]
                vb = pkb[pl.ds(po + 1, LANES)]
                vc = pkb[pl.ds(po + 4 * r, LANES)]
                vd = pkb[pl.ds(po + 4 * r + 1, LANES)]
                bi = (yy * r + xg * LANES + iota) * 16
                plsc.store_scatter(ob, [bi + k], va)
                plsc.store_scatter(ob, [bi + (4 + k)], vb)
                plsc.store_scatter(ob, [bi + (8 + k)], vc)
                plsc.store_scatter(ob, [bi + (12 + k)], vd)
                return cy
            lax.fori_loop(0, yb * 4 * xg_n, il_body, 0)

            pltpu.sync_copy(ob.at[pl.ds(0, yb * r * 16)],
                            dst.at[pl.ds((s * r + y0) * r * 16, yb * r * 16)])

        nt = (ntask + NWORK - 1) // NWORK
        for t in range(nt):
            q = wid + t * NWORK
            if (t + 1) * NWORK <= ntask:
                task_body(q)
            else:
                @pl.when(q < ntask)
                def _():
                    task_body(q)


def _sc_build(p0, p1, p2, p3):
    mesh = plsc.VectorSubcoreMesh(core_axis_name="c", subcore_axis_name="s")
    return pl.kernel(
        _builder_body,
        out_type=tuple(
            jax.ShapeDtypeStruct((NPLANE * r * r * 16,), jnp.int32)
            for r in RES_LIST),
        mesh=mesh,
        compiler_params=pltpu.CompilerParams(
            use_tc_tiling_on_sc=False, needs_layout_passes=False),
        scratch_types=[
            pltpu.VMEM((INB_MAX,), jnp.float32),
            pltpu.VMEM((PKB_MAX,), jnp.int32),
            pltpu.VMEM((OB_WORDS,), jnp.int32),
            pltpu.SemaphoreType.DMA,
        ],
    )(p0, p1, p2, p3)


def _tec_body(x0, x1, x2, tbl0, tbl1, tbl2, tbl3, out,
              xs_v, idx_v, w_v, tap_v, out_v,
              xsem0, xsem1, gsem0, gsem1, osem0, osem1):
    tbls = (tbl0, tbl1, tbl2, tbl3)
    xcoords = (x0, x1, x2)
    xsems = (xsem0, xsem1)
    gsems = (gsem0, gsem1)
    osems = (osem0, osem1)
    wid = lax.axis_index("s") * NCORES + lax.axis_index("c")
    base0 = wid * PPW
    iota = lax.iota(jnp.int32, LANES)
    himask = jnp.full((LANES,), -65536, jnp.int32)

    def fire_xs(ci, b):
        base = base0 + ci * CH
        for d in range(3):
            pltpu.async_copy(xcoords[d].at[pl.ds(base, CH)],
                             xs_v.at[b, pl.ds(d * CH, CH)], xsems[b])

    def wait_xs(b):
        for d in range(3):
            pltpu.make_async_copy(xcoords[d].at[pl.ds(0, CH)],
                                  xs_v.at[b, pl.ds(d * CH, CH)], xsems[b]).wait()

    def compute_idx(b):
        def grp(g, c2):
            g0 = g * LANES
            xd3 = [xs_v[b, pl.ds(d * CH + g0, LANES)] for d in range(3)]
            for l in range(NLVL):
                r = RES_LIST[l]
                st = []
                for d in range(3):
                    t = xd3[d] * float(r) - 0.5
                    i = (t + 1.0).astype(jnp.int32) - 1
                    w1 = t - i.astype(jnp.float32)
                    w0 = 1.0 - w1
                    lo = i < 0
                    hi = i > r - 2
                    ib = jnp.minimum(jnp.maximum(i, 0), r - 2)
                    wA = jnp.where(lo, w1, jnp.where(hi, 0.0, w0))
                    wB = jnp.where(hi, w0, jnp.where(lo, 0.0, w1))
                    st.append((ib, wA, wB))
                for s in range(NPLANE):
                    dx, dy = PLANE_DXY[s]
                    ibx, wxA, wxB = st[dx]
                    iby, wyA, wyB = st[dy]
                    lp = l * NPLANE + s
                    idx_v[b, lp, pl.ds(g0, LANES)] = iby * r + (ibx + s * r * r)
                    w_v[b, lp, 0, pl.ds(g0, LANES)] = wyA * wxA
                    w_v[b, lp, 1, pl.ds(g0, LANES)] = wyA * wxB
                    w_v[b, lp, 2, pl.ds(g0, LANES)] = wyB * wxA
                    w_v[b, lp, 3, pl.ds(g0, LANES)] = wyB * wxB
            return c2
        lax.fori_loop(0, NGRP, grp, 0)

    def fire_gathers(b):
        for l in range(NLVL):
            for s in range(NPLANE):
                lp = l * NPLANE + s
                pltpu.async_copy(tbls[l].at[idx_v.at[b, lp]],
                                 tap_v.at[b, pl.ds(lp * CH, CH), :],
                                 gsems[b])

    def wait_gathers(b):
        for l in range(NLVL):
            for s in range(NPLANE):
                lp = l * NPLANE + s
                pltpu.make_async_copy(tbls[l].at[idx_v.at[b, lp]],
                                      tap_v.at[b, pl.ds(lp * CH, CH), :],
                                      gsems[b]).wait()

    def combine(b):
        tap = tap_v.at[b]
        outb = out_v.at[b]

        def grp(g, c2):
            g0 = g * LANES
            pt = g0 + iota
            pt96 = pt * NOUT
            for lp in range(NLP):
                w00 = w_v[b, lp, 0, pl.ds(g0, LANES)]
                w01 = w_v[b, lp, 1, pl.ds(g0, LANES)]
                w10 = w_v[b, lp, 2, pl.ds(g0, LANES)]
                w11 = w_v[b, lp, 3, pl.ds(g0, LANES)]
                row = pt + lp * CH
                col0 = (lp // NPLANE) * 24 + (lp % NPLANE) * DF
                for k in range(4):
                    wa = plsc.load_gather(tap, [row, jnp.full((LANES,), k, jnp.int32)])
                    wb = plsc.load_gather(tap, [row, jnp.full((LANES,), k + 4, jnp.int32)])
                    wc = plsc.load_gather(tap, [row, jnp.full((LANES,), k + 8, jnp.int32)])
                    wd = plsc.load_gather(tap, [row, jnp.full((LANES,), k + 12, jnp.int32)])
                    alo = plsc.bitcast(wa << 16, jnp.float32)
                    blo = plsc.bitcast(wb << 16, jnp.float32)
                    clo = plsc.bitcast(wc << 16, jnp.float32)
                    dlo = plsc.bitcast(wd << 16, jnp.float32)
                    ahi = plsc.bitcast(wa & himask, jnp.float32)
                    bhi = plsc.bitcast(wb & himask, jnp.float32)
                    chi = plsc.bitcast(wc & himask, jnp.float32)
                    dhi = plsc.bitcast(wd & himask, jnp.float32)
                    even = alo * w00 + blo * w01 + clo * w10 + dlo * w11
                    odd = ahi * w00 + bhi * w01 + chi * w10 + dhi * w11
                    plsc.store_scatter(outb, [pt96 + (col0 + 2 * k)], even)
                    plsc.store_scatter(outb, [pt96 + (col0 + 2 * k + 1)], odd)
            return c2
        lax.fori_loop(0, NGRP, grp, 0)

    def fire_out(ci, b):
        base = base0 + ci * CH
        pltpu.async_copy(out_v.at[b], out.at[pl.ds(base * NOUT, CH * NOUT)],
                         osems[b])

    def wait_out(b):
        pltpu.make_async_copy(out_v.at[b], out.at[pl.ds(0, CH * NOUT)],
                              osems[b]).wait()

    fire_xs(0, 0)
    fire_xs(1, 1)
    wait_xs(0)
    compute_idx(0)
    fire_gathers(0)

    def pair_body(it, carry):
        for b in range(2):
            ci = it * 2 + b
            b1 = 1 - b
            @pl.when(ci + 1 < NCHUNK)
            def _():
                wait_xs(b1)
                compute_idx(b1)
                fire_gathers(b1)

            @pl.when(ci + 2 < NCHUNK)
            def _():
                fire_xs(ci + 2, b)

            @pl.when(ci >= 2)
            def _():
                wait_out(b)

            wait_gathers(b)
            combine(b)
            fire_out(ci, b)
        return carry

    lax.fori_loop(0, NCHUNK // 2, pair_body, 0)
    wait_out(0)
    wait_out(1)


def _sc_lookup(x0, x1, x2, t0, t1, t2, t3):
    mesh = plsc.VectorSubcoreMesh(core_axis_name="c", subcore_axis_name="s")
    return pl.kernel(
        _tec_body,
        out_type=jax.ShapeDtypeStruct((NPTS * NOUT,), jnp.float32),
        mesh=mesh,
        compiler_params=pltpu.CompilerParams(
            use_tc_tiling_on_sc=False, needs_layout_passes=False),
        scratch_types=[
            pltpu.VMEM((2, 3 * CH), jnp.float32),
            pltpu.VMEM((2, NLP, CH), jnp.int32),
            pltpu.VMEM((2, NLP, 4, CH), jnp.float32),
            pltpu.VMEM((2, NLP * CH, 16), jnp.int32),
            pltpu.VMEM((2, CH * NOUT), jnp.float32),
            pltpu.SemaphoreType.DMA,
            pltpu.SemaphoreType.DMA,
            pltpu.SemaphoreType.DMA,
            pltpu.SemaphoreType.DMA,
            pltpu.SemaphoreType.DMA,
            pltpu.SemaphoreType.DMA,
        ],
    )(x0, x1, x2, t0, t1, t2, t3)


def kernel(x, plane0, plane1, plane2, plane3):
    flat_planes = [(p * jnp.float32(1.0000001)).reshape(-1)
                   for p in (plane0, plane1, plane2, plane3)]
    tbl_flat = _sc_build(*flat_planes)
    tbls = [t.reshape(NPLANE * r * r, 16) for t, r in zip(tbl_flat, RES_LIST)]
    flat = _sc_lookup(x[:, 0], x[:, 1], x[:, 2], *tbls)
    return flat.reshape(NPTS, NOUT)
